# baseline (speedup 1.0000x reference)
from functools import partial

import jax
import jax.numpy as jnp
from jax.experimental import pallas as pl
from jax.experimental.pallas import tpu as pltpu

_VMEM_LIMIT_BYTES = 64 * 1024 * 1024


def _patch_matmul_kernel(x_ref, w_ref, o_ref, *, scale):
    wb = w_ref[...].astype(jnp.bfloat16)
    y = jax.lax.dot_general(
        x_ref[...], wb,
        dimension_numbers=(((2,), (1,)), ((0,), (0,))),
        preferred_element_type=jnp.float32,
    )
    o_ref[...] = jnp.maximum(y * scale, 0.0)


def kernel(x, w_unf):
    N, C, D, _ = x.shape
    PP, K, O = w_unf.shape
    f = 4
    P = D // f
    assert PP == P * P and K == C * f * f

    x_unf = (x.reshape(N, C, P, f, P, f)
              .transpose(2, 4, 0, 1, 3, 5)
              .reshape(PP, N, K)
              .astype(jnp.bfloat16))

    scale = 1.0 / float(K) ** 0.5
    bp = 8

    out = pl.pallas_call(
        partial(_patch_matmul_kernel, scale=scale),
        out_shape=jax.ShapeDtypeStruct((PP, N, O), jnp.float32),
        grid=(PP // bp,),
        in_specs=[
            pl.BlockSpec((bp, N, K), lambda pb: (pb, 0, 0)),
            pl.BlockSpec((bp, K, O), lambda pb: (pb, 0, 0)),
        ],
        out_specs=pl.BlockSpec((bp, N, O), lambda pb: (pb, 0, 0)),
        compiler_params=pltpu.CompilerParams(
            dimension_semantics=("parallel",),
            vmem_limit_bytes=_VMEM_LIMIT_BYTES,
        ),
    )(x_unf, w_unf)

    return out.reshape(P, P, N, O).transpose(2, 3, 0, 1)


# speedup vs baseline: 12.9983x; 12.9983x over previous
from functools import partial

import jax
import jax.numpy as jnp
from jax.experimental import pallas as pl
from jax.experimental.pallas import tpu as pltpu

_VMEM_LIMIT_BYTES = 64 * 1024 * 1024


def _loc_fused_kernel(x_ref, w_ref, o_ref, *, scale, n, c, f, p, d):
    s = f * d
    xv = x_ref[...].reshape(n, c, s)
    t1 = jnp.transpose(xv, (0, 2, 1))
    t2 = jnp.transpose(t1.reshape(n * s, c), (1, 0))
    t3 = jnp.transpose(t2.reshape(c, n, s), (0, 2, 1))
    xt = t3.reshape(c * f, d, n)
    for pw in range(p):
        a_t = xt[:, pw * f:(pw + 1) * f, :].reshape(c * f * f, n)
        y = jax.lax.dot_general(
            a_t, w_ref[0, pw],
            dimension_numbers=(((0,), (0,)), ((), ())),
            preferred_element_type=jnp.float32,
        )
        o_ref[0, pw] = jnp.maximum(y * scale, 0.0)


def kernel(x, w_unf):
    N, C, D, _ = x.shape
    PP, K, O = w_unf.shape
    f = 4
    P = D // f
    assert PP == P * P and K == C * f * f

    x5 = x.reshape(N, C, P, 1, f * D)
    w3 = w_unf.reshape(P, P, K, O)
    scale = 1.0 / float(K) ** 0.5

    out = pl.pallas_call(
        partial(_loc_fused_kernel, scale=scale, n=N, c=C, f=f, p=P, d=D),
        out_shape=jax.ShapeDtypeStruct((P, P, N, O), jnp.float32),
        grid=(P,),
        in_specs=[
            pl.BlockSpec((N, C, 1, 1, f * D), lambda ph: (0, 0, ph, 0, 0)),
            pl.BlockSpec((1, P, K, O), lambda ph: (ph, 0, 0, 0)),
        ],
        out_specs=pl.BlockSpec((1, P, N, O), lambda ph: (ph, 0, 0, 0)),
        compiler_params=pltpu.CompilerParams(
            dimension_semantics=("parallel",),
            vmem_limit_bytes=_VMEM_LIMIT_BYTES,
        ),
    )(x5, w3)

    return out.transpose(2, 3, 0, 1)
